# baseline (speedup 1.0000x reference)
import jax
import jax.numpy as jnp
from jax.experimental import pallas as pl
from jax.experimental.pallas import tpu as pltpu


_LANE = 128


def _round_up(x: int, m: int) -> int:
    return ((x + m - 1) // m) * m


def _pool_kernel(x_ref, s_ref, o_ref):
    o_ref[...] = jnp.dot(x_ref[...], s_ref[...],
                         preferred_element_type=jnp.float32)


def _mlp_kernel(f_ref, w1_ref, b1_ref, w2_ref, b2_ref, o_ref):
    fb = f_ref[...].astype(jnp.bfloat16)
    hid = jnp.dot(fb, w1_ref[...], preferred_element_type=jnp.float32)
    hid = jnp.maximum(hid + b1_ref[...], 0.0)
    params = jnp.dot(hid.astype(jnp.bfloat16), w2_ref[...],
                     preferred_element_type=jnp.float32)
    o_ref[...] = params + b2_ref[...]


def kernel(images, w1, b1, w2, b2):
    B, C, H, W = images.shape
    HW = H * W
    hidden = w1.shape[1]
    P = w2.shape[1]
    Hp = _round_up(hidden, _LANE)
    Pp = _round_up(P, _LANE)

    nseg = 16
    rows = (B * C) // nseg
    k = nseg * HW
    x = images.reshape(rows, k)

    seg = jax.lax.broadcasted_iota(jnp.int32, (k, nseg), 0) // HW
    col = jax.lax.broadcasted_iota(jnp.int32, (k, nseg), 1)
    s = (seg == col).astype(jnp.float32)

    num_tiles = 16
    tr = rows // num_tiles

    pooled = pl.pallas_call(
        _pool_kernel,
        out_shape=jax.ShapeDtypeStruct((rows, nseg), jnp.float32),
        grid=(num_tiles,),
        in_specs=[
            pl.BlockSpec((tr, k), lambda i: (i, 0)),
            pl.BlockSpec((k, nseg), lambda i: (0, 0)),
        ],
        out_specs=pl.BlockSpec((tr, nseg), lambda i: (i, 0)),
        compiler_params=pltpu.CompilerParams(
            dimension_semantics=("parallel",),
            vmem_limit_bytes=48 << 20),
    )(x, s)
    feat = pooled.reshape(B, C)

    w1b = (w1.astype(jnp.float32) * (1.0 / HW)).astype(jnp.bfloat16)
    w1b = jnp.pad(w1b, ((0, 0), (0, Hp - hidden)))
    b1p = jnp.pad(b1.astype(jnp.float32).reshape(1, hidden),
                  ((0, 0), (0, Hp - hidden)))
    w2b = jnp.pad(w2.astype(jnp.bfloat16),
                  ((0, Hp - hidden), (0, Pp - P)))
    b2p = jnp.pad(b2.astype(jnp.float32).reshape(1, P),
                  ((0, 0), (0, Pp - P)))

    tb = B // 2
    out = pl.pallas_call(
        _mlp_kernel,
        out_shape=jax.ShapeDtypeStruct((B, Pp), jnp.float32),
        grid=(2,),
        in_specs=[
            pl.BlockSpec((tb, C), lambda i: (i, 0)),
            pl.BlockSpec((C, Hp), lambda i: (0, 0)),
            pl.BlockSpec((1, Hp), lambda i: (0, 0)),
            pl.BlockSpec((Hp, Pp), lambda i: (0, 0)),
            pl.BlockSpec((1, Pp), lambda i: (0, 0)),
        ],
        out_specs=pl.BlockSpec((tb, Pp), lambda i: (i, 0)),
        compiler_params=pltpu.CompilerParams(
            dimension_semantics=("parallel",),
            vmem_limit_bytes=24 << 20),
    )(feat, w1b, b1p, w2b, b2p)
    return out[:, :P]


# speedup vs baseline: 30.2755x; 30.2755x over previous
import jax
import jax.numpy as jnp
from jax.experimental import pallas as pl
from jax.experimental.pallas import tpu as pltpu


_LANE = 128


def _round_up(x: int, m: int) -> int:
    return ((x + m - 1) // m) * m


def _fused_kernel(x_ref, w1_ref, b1_ref, w2_ref, b2_ref, o_ref, acc_ref,
                  *, num_hw_steps, inv_hw):
    i = pl.program_id(1)
    part = jnp.sum(x_ref[...], axis=0)

    @pl.when(i == 0)
    def _init():
        acc_ref[...] = part

    @pl.when(i > 0)
    def _accum():
        acc_ref[...] += part

    @pl.when(i == num_hw_steps - 1)
    def _head():
        feat = acc_ref[...] * inv_hw
        hid = jnp.dot(feat, w1_ref[...], preferred_element_type=jnp.float32)
        hid = jnp.maximum(hid + b1_ref[...], 0.0)
        params = jnp.dot(hid, w2_ref[...], preferred_element_type=jnp.float32)
        o_ref[...] = params + b2_ref[...]


def kernel(images, w1, b1, w2, b2):
    B, C, H, W = images.shape
    HW = H * W
    hidden = w1.shape[1]
    P = w2.shape[1]
    Hp = _round_up(hidden, _LANE)
    Pp = _round_up(P, _LANE)

    x = jnp.transpose(images, (2, 3, 0, 1)).reshape(HW, B, C)

    w1p = jnp.pad(w1.astype(jnp.float32), ((0, 0), (0, Hp - hidden)))
    b1p = jnp.pad(b1.astype(jnp.float32).reshape(1, hidden),
                  ((0, 0), (0, Hp - hidden)))
    w2p = jnp.pad(w2.astype(jnp.float32), ((0, Hp - hidden), (0, Pp - P)))
    b2p = jnp.pad(b2.astype(jnp.float32).reshape(1, P),
                  ((0, 0), (0, Pp - P)))

    num_hw_steps = 7
    hw_chunk = HW // num_hw_steps
    tb = B // 2

    import functools
    body = functools.partial(_fused_kernel, num_hw_steps=num_hw_steps,
                             inv_hw=1.0 / HW)

    out = pl.pallas_call(
        body,
        out_shape=jax.ShapeDtypeStruct((B, Pp), jnp.float32),
        grid=(2, num_hw_steps),
        in_specs=[
            pl.BlockSpec((hw_chunk, tb, C), lambda b, i: (i, b, 0)),
            pl.BlockSpec((C, Hp), lambda b, i: (0, 0)),
            pl.BlockSpec((1, Hp), lambda b, i: (0, 0)),
            pl.BlockSpec((Hp, Pp), lambda b, i: (0, 0)),
            pl.BlockSpec((1, Pp), lambda b, i: (0, 0)),
        ],
        out_specs=pl.BlockSpec((tb, Pp), lambda b, i: (b, 0)),
        scratch_shapes=[pltpu.VMEM((tb, C), jnp.float32)],
        compiler_params=pltpu.CompilerParams(
            dimension_semantics=("parallel", "arbitrary"),
            vmem_limit_bytes=48 << 20),
    )(x, w1p, b1p, w2p, b2p)
    return out[:, :P]


# speedup vs baseline: 30.4352x; 1.0053x over previous
import jax
import jax.numpy as jnp
from jax.experimental import pallas as pl
from jax.experimental.pallas import tpu as pltpu


_LANE = 128


def _round_up(x: int, m: int) -> int:
    return ((x + m - 1) // m) * m


def _fused_kernel(x_ref, w1_ref, b1_ref, w2_ref, b2_ref, o_ref, acc_ref,
                  *, num_hw_steps, inv_hw):
    i = pl.program_id(1)
    part = jnp.sum(x_ref[...], axis=0)

    @pl.when(i == 0)
    def _init():
        acc_ref[...] = part

    @pl.when(i > 0)
    def _accum():
        acc_ref[...] += part

    @pl.when(i == num_hw_steps - 1)
    def _head():
        feat = acc_ref[...] * inv_hw
        hid = jnp.dot(feat, w1_ref[...], preferred_element_type=jnp.float32)
        hid = jnp.maximum(hid + b1_ref[...], 0.0)
        params = jnp.dot(hid, w2_ref[...], preferred_element_type=jnp.float32)
        o_ref[...] = params + b2_ref[...]


def kernel(images, w1, b1, w2, b2):
    B, C, H, W = images.shape
    HW = H * W
    hidden = w1.shape[1]
    P = w2.shape[1]

    x = jnp.transpose(images, (2, 3, 0, 1)).reshape(HW, B, C)

    num_hw_steps = 7
    hw_chunk = HW // num_hw_steps
    tb = B // 2

    import functools
    body = functools.partial(_fused_kernel, num_hw_steps=num_hw_steps,
                             inv_hw=1.0 / HW)

    out = pl.pallas_call(
        body,
        out_shape=jax.ShapeDtypeStruct((B, P), jnp.float32),
        grid=(2, num_hw_steps),
        in_specs=[
            pl.BlockSpec((hw_chunk, tb, C), lambda b, i: (i, b, 0)),
            pl.BlockSpec((C, hidden), lambda b, i: (0, 0)),
            pl.BlockSpec((1, hidden), lambda b, i: (0, 0)),
            pl.BlockSpec((hidden, P), lambda b, i: (0, 0)),
            pl.BlockSpec((1, P), lambda b, i: (0, 0)),
        ],
        out_specs=pl.BlockSpec((tb, P), lambda b, i: (b, 0)),
        scratch_shapes=[pltpu.VMEM((tb, C), jnp.float32)],
        compiler_params=pltpu.CompilerParams(
            dimension_semantics=("parallel", "arbitrary"),
            vmem_limit_bytes=48 << 20),
    )(x, w1, b1, w2, b2)
    return out


# speedup vs baseline: 34.3275x; 1.1279x over previous
import functools

import jax
import jax.numpy as jnp
from jax.experimental import pallas as pl
from jax.experimental.pallas import tpu as pltpu


def _pool_mm_kernel(x_ref, w1_ref, o_ref, acc_ref, *, num_hw_steps, inv_hw):
    i = pl.program_id(1)
    part = jnp.sum(x_ref[...], axis=0)

    @pl.when(i == 0)
    def _init():
        acc_ref[...] = part

    @pl.when(i > 0)
    def _accum():
        acc_ref[...] += part

    @pl.when(i == num_hw_steps - 1)
    def _matmul():
        feat = acc_ref[...] * inv_hw
        o_ref[0] = jnp.dot(feat, w1_ref[...],
                           preferred_element_type=jnp.float32)


def _head_kernel(h_ref, b1_ref, w2_ref, b2_ref, o_ref):
    hid = jnp.maximum(h_ref[0] + h_ref[1] + b1_ref[...], 0.0)
    params = jnp.dot(hid, w2_ref[...], preferred_element_type=jnp.float32)
    o_ref[...] = params + b2_ref[...]


def kernel(images, w1, b1, w2, b2):
    B, C, H, W = images.shape
    HW = H * W
    hidden = w1.shape[1]
    P = w2.shape[1]

    x = jnp.transpose(images, (2, 3, 0, 1)).reshape(HW, B, C)

    num_hw_steps = 7
    hw_chunk = HW // num_hw_steps
    ch = C // 2

    body = functools.partial(_pool_mm_kernel, num_hw_steps=num_hw_steps,
                             inv_hw=1.0 / HW)

    hid_parts = pl.pallas_call(
        body,
        out_shape=jax.ShapeDtypeStruct((2, B, hidden), jnp.float32),
        grid=(2, num_hw_steps),
        in_specs=[
            pl.BlockSpec((hw_chunk, B, ch), lambda c, i: (i, 0, c)),
            pl.BlockSpec((ch, hidden), lambda c, i: (c, 0)),
        ],
        out_specs=pl.BlockSpec((1, B, hidden), lambda c, i: (c, 0, 0)),
        scratch_shapes=[pltpu.VMEM((B, ch), jnp.float32)],
        compiler_params=pltpu.CompilerParams(
            dimension_semantics=("parallel", "arbitrary"),
            vmem_limit_bytes=48 << 20),
    )(x, w1)

    tb = B // 2
    out = pl.pallas_call(
        _head_kernel,
        out_shape=jax.ShapeDtypeStruct((B, P), jnp.float32),
        grid=(2,),
        in_specs=[
            pl.BlockSpec((2, tb, hidden), lambda b: (0, b, 0)),
            pl.BlockSpec((1, hidden), lambda b: (0, 0)),
            pl.BlockSpec((hidden, P), lambda b: (0, 0)),
            pl.BlockSpec((1, P), lambda b: (0, 0)),
        ],
        out_specs=pl.BlockSpec((tb, P), lambda b: (b, 0)),
        compiler_params=pltpu.CompilerParams(
            dimension_semantics=("parallel",),
            vmem_limit_bytes=24 << 20),
    )(hid_parts, b1, w2, b2)
    return out


# speedup vs baseline: 34.5345x; 1.0060x over previous
import functools

import jax
import jax.numpy as jnp
from jax.experimental import pallas as pl
from jax.experimental.pallas import tpu as pltpu


def _pool_mm_kernel(x_ref, w1_ref, o_ref, acc_ref, *, num_hw_steps, inv_hw):
    i = pl.program_id(1)
    part = jnp.sum(x_ref[...], axis=0)

    @pl.when(i == 0)
    def _init():
        acc_ref[...] = part

    @pl.when(i > 0)
    def _accum():
        acc_ref[...] += part

    @pl.when(i == num_hw_steps - 1)
    def _matmul():
        feat = acc_ref[...] * inv_hw
        o_ref[0] = jnp.dot(feat, w1_ref[...],
                           preferred_element_type=jnp.float32
                           ).astype(jnp.bfloat16)


def _head_kernel(h_ref, b1_ref, w2_ref, b2_ref, o_ref):
    hid = h_ref[0].astype(jnp.float32) + h_ref[1].astype(jnp.float32)
    hid = jnp.maximum(hid + b1_ref[...], 0.0)
    params = jnp.dot(hid, w2_ref[...], preferred_element_type=jnp.float32)
    o_ref[...] = params + b2_ref[...]


def kernel(images, w1, b1, w2, b2):
    B, C, H, W = images.shape
    HW = H * W
    hidden = w1.shape[1]
    P = w2.shape[1]

    x = jnp.transpose(images, (2, 3, 0, 1)).reshape(HW, B, C)

    num_hw_steps = 7
    hw_chunk = HW // num_hw_steps
    ch = C // 2

    body = functools.partial(_pool_mm_kernel, num_hw_steps=num_hw_steps,
                             inv_hw=1.0 / HW)

    hid_parts = pl.pallas_call(
        body,
        out_shape=jax.ShapeDtypeStruct((2, B, hidden), jnp.bfloat16),
        grid=(2, num_hw_steps),
        in_specs=[
            pl.BlockSpec((hw_chunk, B, ch), lambda c, i: (i, 0, c)),
            pl.BlockSpec((ch, hidden), lambda c, i: (c, 0)),
        ],
        out_specs=pl.BlockSpec((1, B, hidden), lambda c, i: (c, 0, 0)),
        scratch_shapes=[pltpu.VMEM((B, ch), jnp.float32)],
        compiler_params=pltpu.CompilerParams(
            dimension_semantics=("parallel", "arbitrary"),
            vmem_limit_bytes=48 << 20),
    )(x, w1)

    tb = B // 2
    out = pl.pallas_call(
        _head_kernel,
        out_shape=jax.ShapeDtypeStruct((B, P), jnp.float32),
        grid=(2,),
        in_specs=[
            pl.BlockSpec((2, tb, hidden), lambda b: (0, b, 0)),
            pl.BlockSpec((1, hidden), lambda b: (0, 0)),
            pl.BlockSpec((hidden, P), lambda b: (0, 0)),
            pl.BlockSpec((1, P), lambda b: (0, 0)),
        ],
        out_specs=pl.BlockSpec((tb, P), lambda b: (b, 0)),
        compiler_params=pltpu.CompilerParams(
            dimension_semantics=("parallel",),
            vmem_limit_bytes=24 << 20),
    )(hid_parts, b1, w2, b2)
    return out
